# baseline (speedup 1.0000x reference)
import jax
import jax.numpy as jnp
from jax.experimental import pallas as pl
from jax.experimental.pallas import tpu as pltpu
from jax.experimental.pallas import tpu_sc as plsc

VOCAB = 100000
DIM = 128
L = 200
TEXT_OUT = 512
N_PROMPTS = 2
K = L * DIM

GATHER_WINDOW = 128
N_IDX_PAD = 512
K_TILE = 6400


def _sc_gather(embeddings, idx2d):
    n_rows = idx2d.shape[1]
    mesh = plsc.VectorSubcoreMesh(core_axis_name="core",
                                  subcore_axis_name="subcore")

    @pl.kernel(out_type=jax.ShapeDtypeStruct((n_rows, DIM), embeddings.dtype),
               mesh=mesh)
    def gather_kernel(x_hbm, i_hbm, o_hbm):
        def body(i_vmem, o_vmem):
            pltpu.sync_copy(x_hbm.at[i_vmem.at[0]], o_vmem)

        pltpu.emit_pipeline(
            body,
            grid=(n_rows // GATHER_WINDOW,),
            in_specs=[pl.BlockSpec((1, GATHER_WINDOW),
                                   index_map=lambda i: (0, i))],
            out_specs=[pl.BlockSpec((GATHER_WINDOW, DIM),
                                    index_map=lambda i: (i, 0))],
            core_axis_name="subcore",
            dimension_semantics=(pltpu.PARALLEL,),
        )(i_hbm, o_hbm)

    return gather_kernel(embeddings, idx2d)


def _mm_body(p_ref, v_ref, w_ref, o_ref):
    k = pl.program_id(0)

    @pl.when(k == 0)
    def _():
        o_ref[...] = jnp.zeros_like(o_ref)

    p = p_ref[...] + v_ref[...]
    o_ref[...] += jnp.dot(p, w_ref[...], preferred_element_type=jnp.float32)


def _fused_matmul(p, v, w):
    return pl.pallas_call(
        _mm_body,
        grid=(K // K_TILE,),
        in_specs=[
            pl.BlockSpec((N_PROMPTS, K_TILE), lambda k: (0, k)),
            pl.BlockSpec((N_PROMPTS, K_TILE), lambda k: (0, k)),
            pl.BlockSpec((K_TILE, TEXT_OUT), lambda k: (k, 0)),
        ],
        out_specs=pl.BlockSpec((N_PROMPTS, TEXT_OUT), lambda k: (0, 0)),
        out_shape=jax.ShapeDtypeStruct((N_PROMPTS, TEXT_OUT), jnp.float32),
    )(p, v, w)


def kernel(vis_features_first, vis_features_second, inputs_first,
           inputs_second, embeddings, W_text):
    pad = jnp.zeros((N_IDX_PAD - N_PROMPTS * L,), jnp.int32)
    idx = jnp.concatenate([inputs_first.astype(jnp.int32),
                           inputs_second.astype(jnp.int32), pad])
    idx2d = idx.reshape(1, N_IDX_PAD)
    gathered = _sc_gather(embeddings, idx2d)
    p = gathered[:N_PROMPTS * L].reshape(N_PROMPTS, K)
    v = jnp.concatenate([vis_features_first, vis_features_second], axis=0)
    out = _fused_matmul(p, v, W_text)
    return (out[0:1], out[1:2])


# speedup vs baseline: 1.0168x; 1.0168x over previous
import jax
import jax.numpy as jnp
from jax.experimental import pallas as pl
from jax.experimental.pallas import tpu as pltpu
from jax.experimental.pallas import tpu_sc as plsc

VOCAB = 100000
DIM = 128
L = 200
TEXT_OUT = 512
N_PROMPTS = 2
K = L * DIM

GATHER_WINDOW = 128
N_IDX_PAD = 512
K_TILE = 2560


def _sc_gather(embeddings, idx2d):
    n_rows = idx2d.shape[1]
    mesh = plsc.VectorSubcoreMesh(core_axis_name="core",
                                  subcore_axis_name="subcore")

    @pl.kernel(out_type=jax.ShapeDtypeStruct((n_rows, DIM), embeddings.dtype),
               mesh=mesh)
    def gather_kernel(x_hbm, i_hbm, o_hbm):
        def body(i_vmem, o_vmem):
            pltpu.sync_copy(x_hbm.at[i_vmem.at[0]], o_vmem)

        pltpu.emit_pipeline(
            body,
            grid=(n_rows // GATHER_WINDOW,),
            in_specs=[pl.BlockSpec((1, GATHER_WINDOW),
                                   index_map=lambda i: (0, i))],
            out_specs=[pl.BlockSpec((GATHER_WINDOW, DIM),
                                    index_map=lambda i: (i, 0))],
            core_axis_name="subcore",
            dimension_semantics=(pltpu.PARALLEL,),
        )(i_hbm, o_hbm)

    return gather_kernel(embeddings, idx2d)


def _mm_body(p_ref, v_ref, w_ref, o_ref):
    k = pl.program_id(0)

    @pl.when(k == 0)
    def _():
        o_ref[...] = jnp.zeros_like(o_ref)

    p = p_ref[...] + v_ref[...]
    o_ref[...] += jnp.dot(p, w_ref[...], preferred_element_type=jnp.float32)


def _fused_matmul(p, v, w):
    return pl.pallas_call(
        _mm_body,
        grid=(K // K_TILE,),
        in_specs=[
            pl.BlockSpec((N_PROMPTS, K_TILE), lambda k: (0, k)),
            pl.BlockSpec((N_PROMPTS, K_TILE), lambda k: (0, k)),
            pl.BlockSpec((K_TILE, TEXT_OUT), lambda k: (k, 0)),
        ],
        out_specs=pl.BlockSpec((N_PROMPTS, TEXT_OUT), lambda k: (0, 0)),
        out_shape=jax.ShapeDtypeStruct((N_PROMPTS, TEXT_OUT), jnp.float32),
    )(p, v, w)


def kernel(vis_features_first, vis_features_second, inputs_first,
           inputs_second, embeddings, W_text):
    pad = jnp.zeros((N_IDX_PAD - N_PROMPTS * L,), jnp.int32)
    idx = jnp.concatenate([inputs_first.astype(jnp.int32),
                           inputs_second.astype(jnp.int32), pad])
    idx2d = idx.reshape(1, N_IDX_PAD)
    gathered = _sc_gather(embeddings, idx2d)
    p = gathered[:N_PROMPTS * L].reshape(N_PROMPTS, K)
    v = jnp.concatenate([vis_features_first, vis_features_second], axis=0)
    out = _fused_matmul(p, v, W_text)
    return (out[0:1], out[1:2])


# speedup vs baseline: 1.1676x; 1.1484x over previous
import functools

import jax
import jax.numpy as jnp
from jax import lax
from jax.experimental import pallas as pl
from jax.experimental.pallas import tpu as pltpu
from jax.experimental.pallas import tpu_sc as plsc

VOCAB = 100000
DIM = 128
L = 200
TEXT_OUT = 512
N_PROMPTS = 2
K = L * DIM

SC_CORES = 2
SC_SUBCORES = 16
SC_WORKERS = SC_CORES * SC_SUBCORES
N_IDX_PAD = 512
B_PER_W = N_IDX_PAD // SC_WORKERS
K_TILE = 3200


def _sc_gather(embeddings, idx):
    mesh = plsc.VectorSubcoreMesh(core_axis_name="c", subcore_axis_name="s")

    @functools.partial(
        pl.kernel, mesh=mesh,
        out_type=jax.ShapeDtypeStruct((N_IDX_PAD, DIM), embeddings.dtype),
        scratch_types=[
            pltpu.VMEM((B_PER_W,), jnp.int32),
            pltpu.VMEM((B_PER_W, DIM), jnp.float32),
            pltpu.SemaphoreType.DMA,
        ],
    )
    def gather_kernel(table_hbm, idx_hbm, out_hbm, idx_v, rows_v, sem):
        wid = lax.axis_index("s") * SC_CORES + lax.axis_index("c")
        base = wid * B_PER_W
        pltpu.sync_copy(idx_hbm.at[pl.ds(base, B_PER_W)], idx_v)
        pltpu.async_copy(table_hbm.at[idx_v], rows_v, sem).wait()
        pltpu.sync_copy(rows_v, out_hbm.at[pl.ds(base, B_PER_W)])

    return gather_kernel(embeddings, idx)


def _mm_body(p_ref, v_ref, w_ref, o_ref):
    k = pl.program_id(0)

    @pl.when(k == 0)
    def _():
        o_ref[...] = jnp.zeros_like(o_ref)

    p = p_ref[...] + v_ref[...]
    o_ref[...] += jnp.dot(p, w_ref[...], preferred_element_type=jnp.float32)


def _fused_matmul(p, v, w):
    return pl.pallas_call(
        _mm_body,
        grid=(K // K_TILE,),
        in_specs=[
            pl.BlockSpec((N_PROMPTS, K_TILE), lambda k: (0, k)),
            pl.BlockSpec((N_PROMPTS, K_TILE), lambda k: (0, k)),
            pl.BlockSpec((K_TILE, TEXT_OUT), lambda k: (k, 0)),
        ],
        out_specs=pl.BlockSpec((N_PROMPTS, TEXT_OUT), lambda k: (0, 0)),
        out_shape=jax.ShapeDtypeStruct((N_PROMPTS, TEXT_OUT), jnp.float32),
    )(p, v, w)


def kernel(vis_features_first, vis_features_second, inputs_first,
           inputs_second, embeddings, W_text):
    pad = jnp.zeros((N_IDX_PAD - N_PROMPTS * L,), jnp.int32)
    idx = jnp.concatenate([inputs_first.astype(jnp.int32),
                           inputs_second.astype(jnp.int32), pad])
    gathered = _sc_gather(embeddings, idx)
    p = gathered[:N_PROMPTS * L].reshape(N_PROMPTS, K)
    v = jnp.concatenate([vis_features_first, vis_features_second], axis=0)
    out = _fused_matmul(p, v, W_text)
    return (out[0:1], out[1:2])


# speedup vs baseline: 1.1738x; 1.0053x over previous
import functools

import jax
import jax.numpy as jnp
from jax import lax
from jax.experimental import pallas as pl
from jax.experimental.pallas import tpu as pltpu
from jax.experimental.pallas import tpu_sc as plsc

VOCAB = 100000
DIM = 128
L = 200
TEXT_OUT = 512
N_PROMPTS = 2
K = L * DIM

SC_CORES = 2
SC_SUBCORES = 16
SC_WORKERS = SC_CORES * SC_SUBCORES
N_IDX_PAD = 512
B_PER_W = N_IDX_PAD // SC_WORKERS
K_TILE = 3200


def _sc_gather(embeddings, idx):
    mesh = plsc.VectorSubcoreMesh(core_axis_name="c", subcore_axis_name="s")

    @functools.partial(
        pl.kernel, mesh=mesh,
        out_type=jax.ShapeDtypeStruct((N_IDX_PAD, DIM), embeddings.dtype),
        scratch_types=[
            pltpu.VMEM((B_PER_W,), jnp.int32),
            pltpu.VMEM((B_PER_W, DIM), jnp.float32),
            pltpu.SemaphoreType.DMA,
        ],
    )
    def gather_kernel(table_hbm, idx_hbm, out_hbm, idx_v, rows_v, sem):
        wid = lax.axis_index("s") * SC_CORES + lax.axis_index("c")
        base = wid * B_PER_W
        pltpu.sync_copy(idx_hbm.at[pl.ds(base, B_PER_W)], idx_v)
        pltpu.async_copy(table_hbm.at[idx_v], rows_v, sem).wait()
        pltpu.sync_copy(rows_v, out_hbm.at[pl.ds(base, B_PER_W)])

    return gather_kernel(embeddings, idx)


def _mm_body(p_ref, v1_ref, v2_ref, w_ref, o1_ref, o2_ref):
    k = pl.program_id(0)

    @pl.when(k == 0)
    def _():
        o1_ref[...] = jnp.zeros_like(o1_ref)
        o2_ref[...] = jnp.zeros_like(o2_ref)

    w = w_ref[...]
    o1_ref[...] += jnp.dot(p_ref[0:1] + v1_ref[...], w,
                           preferred_element_type=jnp.float32)
    o2_ref[...] += jnp.dot(p_ref[1:2] + v2_ref[...], w,
                           preferred_element_type=jnp.float32)


def _fused_matmul(p, v1, v2, w):
    return pl.pallas_call(
        _mm_body,
        grid=(K // K_TILE,),
        in_specs=[
            pl.BlockSpec((N_PROMPTS, K_TILE), lambda k: (0, k)),
            pl.BlockSpec((1, K_TILE), lambda k: (0, k)),
            pl.BlockSpec((1, K_TILE), lambda k: (0, k)),
            pl.BlockSpec((K_TILE, TEXT_OUT), lambda k: (k, 0)),
        ],
        out_specs=[
            pl.BlockSpec((1, TEXT_OUT), lambda k: (0, 0)),
            pl.BlockSpec((1, TEXT_OUT), lambda k: (0, 0)),
        ],
        out_shape=[
            jax.ShapeDtypeStruct((1, TEXT_OUT), jnp.float32),
            jax.ShapeDtypeStruct((1, TEXT_OUT), jnp.float32),
        ],
    )(p, v1, v2, w)


def kernel(vis_features_first, vis_features_second, inputs_first,
           inputs_second, embeddings, W_text):
    pad = jnp.zeros((N_IDX_PAD - N_PROMPTS * L,), jnp.int32)
    idx = jnp.concatenate([inputs_first.astype(jnp.int32),
                           inputs_second.astype(jnp.int32), pad])
    gathered = _sc_gather(embeddings, idx)
    p = gathered[:N_PROMPTS * L].reshape(N_PROMPTS, K)
    return _fused_matmul(p, vis_features_first, vis_features_second, W_text)


# speedup vs baseline: 1.2128x; 1.0332x over previous
import functools

import jax
import jax.numpy as jnp
from jax import lax
from jax.experimental import pallas as pl
from jax.experimental.pallas import tpu as pltpu
from jax.experimental.pallas import tpu_sc as plsc

VOCAB = 100000
DIM = 128
L = 200
TEXT_OUT = 512
N_PROMPTS = 2
K = L * DIM

SC_CORES = 2
SC_SUBCORES = 16
SC_WORKERS = SC_CORES * SC_SUBCORES
N_IDX_PAD = 512
B_PER_W = N_IDX_PAD // SC_WORKERS
K_TILE = 3200


def _sc_gather(embeddings, idx):
    mesh = plsc.VectorSubcoreMesh(core_axis_name="c", subcore_axis_name="s")

    @functools.partial(
        pl.kernel, mesh=mesh,
        out_type=jax.ShapeDtypeStruct((N_IDX_PAD, DIM), embeddings.dtype),
        scratch_types=[
            pltpu.VMEM((B_PER_W,), jnp.int32),
            pltpu.VMEM((B_PER_W, DIM), jnp.float32),
            pltpu.SemaphoreType.DMA,
        ],
    )
    def gather_kernel(table_hbm, idx_hbm, out_hbm, idx_v, rows_v, sem):
        wid = lax.axis_index("s") * SC_CORES + lax.axis_index("c")
        base = wid * B_PER_W
        pltpu.sync_copy(idx_hbm.at[pl.ds(base, B_PER_W)], idx_v)
        pltpu.async_copy(table_hbm.at[idx_v], rows_v, sem).wait()
        pltpu.sync_copy(rows_v, out_hbm.at[pl.ds(base, B_PER_W)])

    return gather_kernel(embeddings, idx)


ROWS_PER_TILE = K_TILE // DIM


def _mm_body(g_ref, v1_ref, v2_ref, w_ref, o1_ref, o2_ref):
    k = pl.program_id(0)

    @pl.when(k == 0)
    def _():
        o1_ref[...] = jnp.zeros_like(o1_ref)
        o2_ref[...] = jnp.zeros_like(o2_ref)

    w = w_ref[...]
    p1 = g_ref[pl.ds(k * ROWS_PER_TILE, ROWS_PER_TILE), :]
    p2 = g_ref[pl.ds(L + k * ROWS_PER_TILE, ROWS_PER_TILE), :]
    p1 = p1.reshape(1, K_TILE) + v1_ref[...]
    p2 = p2.reshape(1, K_TILE) + v2_ref[...]
    o1_ref[...] += jnp.dot(p1, w, preferred_element_type=jnp.float32)
    o2_ref[...] += jnp.dot(p2, w, preferred_element_type=jnp.float32)


def _fused_matmul(gathered, v1, v2, w):
    return pl.pallas_call(
        _mm_body,
        grid=(K // K_TILE,),
        in_specs=[
            pl.BlockSpec((N_IDX_PAD, DIM), lambda k: (0, 0)),
            pl.BlockSpec((1, K_TILE), lambda k: (0, k)),
            pl.BlockSpec((1, K_TILE), lambda k: (0, k)),
            pl.BlockSpec((K_TILE, TEXT_OUT), lambda k: (k, 0)),
        ],
        out_specs=[
            pl.BlockSpec((1, TEXT_OUT), lambda k: (0, 0)),
            pl.BlockSpec((1, TEXT_OUT), lambda k: (0, 0)),
        ],
        out_shape=[
            jax.ShapeDtypeStruct((1, TEXT_OUT), jnp.float32),
            jax.ShapeDtypeStruct((1, TEXT_OUT), jnp.float32),
        ],
    )(gathered, v1, v2, w)


def kernel(vis_features_first, vis_features_second, inputs_first,
           inputs_second, embeddings, W_text):
    pad = jnp.zeros((N_IDX_PAD - N_PROMPTS * L,), jnp.int32)
    idx = jnp.concatenate([inputs_first.astype(jnp.int32),
                           inputs_second.astype(jnp.int32), pad])
    gathered = _sc_gather(embeddings, idx)
    return _fused_matmul(gathered, vis_features_first, vis_features_second,
                         W_text)
